# baseline (speedup 1.0000x reference)
import functools

import jax
import jax.numpy as jnp
from jax import lax
from jax.experimental import pallas as pl
from jax.experimental.pallas import tpu as pltpu
from jax.experimental.pallas import tpu_sc as plsc

_B = 16384
_D = 32
_NC = 2
_NS = 16
_NW = _NC * _NS
_BPW = _B // _NW
_CHUNK = 128
_NCHUNK = _BPW // _CHUNK
_LANES = 16
_RING = 8


def _gmf_body(uix_hbm, iix_hbm, ut_hbm, it_hbm, out_hbm,
              uidx_v, iidx_v, slab, outT, sems):
    wid = lax.axis_index("s") * _NC + lax.axis_index("c")
    iota = lax.iota(jnp.int32, _LANES)

    pltpu.sync_copy(uix_hbm.at[pl.ds(wid * _NCHUNK, _NCHUNK)], uidx_v)
    pltpu.sync_copy(iix_hbm.at[pl.ds(wid * _NCHUNK, _NCHUNK)], iidx_v)

    def read_idx(ref, i):
        c0 = (i % _CHUNK) >> 4 << 4
        vec = ref[i // _CHUNK, pl.ds(c0, _LANES)]
        lane = i & (_LANES - 1)
        return jnp.sum(jnp.where(iota == lane, vec, 0), axis=0)

    def fire(i, slot):
        ru = read_idx(uidx_v, i)
        ri = read_idx(iidx_v, i)
        ug = pl.multiple_of((ru >> 7) * _CHUNK, _CHUNK)
        ig = pl.multiple_of((ri >> 7) * _CHUNK, _CHUNK)
        pltpu.async_copy(
            ut_hbm.at[:, pl.ds(ug, _CHUNK)], slab.at[slot, 0], sems[slot])
        pltpu.async_copy(
            it_hbm.at[:, pl.ds(ig, _CHUNK)], slab.at[slot, 1], sems[slot])

    def wait_pair(slot):
        pltpu.make_async_copy(
            ut_hbm.at[:, pl.ds(0, 2 * _CHUNK)], slab.at[slot], sems[slot]).wait()

    def extract(slot, i):
        ru = read_idx(uidx_v, i)
        ri = read_idx(iidx_v, i)
        cu = jnp.full((_LANES,), ru & 127, jnp.int32)
        ci = jnp.full((_LANES,), ri & 127, jnp.int32)
        icol = jnp.full((_LANES,), i, jnp.int32)
        for h in range(2):
            rows = iota + h * _LANES
            u = plsc.load_gather(slab.at[slot, 0], [rows, cu])
            v = plsc.load_gather(slab.at[slot, 1], [rows, ci])
            plsc.store_scatter(outT, [rows, icol], u * v)

    for p in range(_RING):
        fire(p, p)

    def group(g, carry):
        for p in range(_RING):
            i = g * _RING + p
            wait_pair(p)
            extract(p, i)
            @pl.when(i + _RING < _BPW)
            def _():
                fire(i + _RING, p)
        return carry

    lax.fori_loop(0, _BPW // _RING, group, 0)

    pltpu.sync_copy(outT, out_hbm.at[:, pl.ds(wid * _BPW, _BPW)])


@functools.partial(
    pl.kernel,
    out_type=jax.ShapeDtypeStruct((_D, _B), jnp.float32),
    mesh=plsc.VectorSubcoreMesh(core_axis_name="c", subcore_axis_name="s"),
    compiler_params=pltpu.CompilerParams(
        use_tc_tiling_on_sc=True, needs_layout_passes=False),
    scratch_types=[
        pltpu.VMEM((_NCHUNK, _CHUNK), jnp.int32),
        pltpu.VMEM((_NCHUNK, _CHUNK), jnp.int32),
        pltpu.VMEM((_RING, 2, _D, _CHUNK), jnp.float32),
        pltpu.VMEM((_D, _BPW), jnp.float32),
        [pltpu.SemaphoreType.DMA] * _RING,
    ],
)
def _gmf(uix_hbm, iix_hbm, ut_hbm, it_hbm, out_hbm,
         uidx_v, iidx_v, slab, outT, sems):
    _gmf_body(uix_hbm, iix_hbm, ut_hbm, it_hbm, out_hbm,
              uidx_v, iidx_v, slab, outT, sems)


def kernel(user, item, user_table, item_table):
    uix = user.astype(jnp.int32).reshape(_B // _CHUNK, _CHUNK)
    iix = item.astype(jnp.int32).reshape(_B // _CHUNK, _CHUNK)
    out_t = _gmf(uix, iix, user_table.T, item_table.T)
    return out_t.T
